# baseline (speedup 1.0000x reference)
import functools

import jax
import jax.numpy as jnp
from jax import lax
from jax.experimental import pallas as pl
from jax.experimental.pallas import tpu as pltpu
from jax.experimental.pallas import tpu_sc as plsc

_B = 16384
_F = 26
_D = 32
_NW = 32
_RPW = _B // _NW
_CHUNK = 16
_NCHUNK = _RPW // _CHUNK
_IDX_PER_CHUNK = _CHUNK * _F
_G = 104
_GPC = _IDX_PER_CHUNK // _G
_GPW = _RPW * _F // _G
_OFFP = 512


def _fm_body(xhi_hbm, xfull_hbm, off_hbm, lw_hbm, emb_hbm, out_hbm,
             idx_v, xfull_v, rows_v, lin_v, off_v, out_v, sem):
    wid = lax.axis_index("s") * 2 + lax.axis_index("c")

    pltpu.sync_copy(xhi_hbm.at[wid], idx_v)
    pltpu.sync_copy(xfull_hbm.at[wid], xfull_v)

    lane = lax.broadcasted_iota(jnp.int32, (16,), 0)
    lmask = jnp.where(lane < _F - 16, 1.0, 0.0)

    def chunk_body(c, _):
        copies = [pltpu.async_copy(off_hbm.at[wid, c], off_v, sem)]
        for j in range(_GPC):
            g = c * _GPC + j
            copies.append(pltpu.async_copy(
                emb_hbm.at[idx_v.at[g]], rows_v.at[pl.ds(j * _G, _G)], sem))
            copies.append(pltpu.async_copy(
                lw_hbm.at[xfull_v.at[g]], lin_v.at[pl.ds(j * _G, _G)], sem))
        for cp in copies:
            cp.wait()

        def row_body(i, acc):
            base = i * _F
            offs0 = off_v[pl.ds(base, 16)]
            offs1 = off_v[pl.ds(base + 16, 16)]
            s0 = jnp.zeros((16,), jnp.float32)
            s1 = jnp.zeros((16,), jnp.float32)
            q0 = jnp.zeros((16,), jnp.float32)
            q1 = jnp.zeros((16,), jnp.float32)
            for f in range(_F):
                o = offs0[f] if f < 16 else offs1[f - 16]
                v0 = rows_v[base + f, pl.ds(o, 16)]
                v1 = rows_v[base + f, pl.ds(o + 16, 16)]
                s0 = s0 + v0
                q0 = q0 + v0 * v0
                s1 = s1 + v1
                q1 = q1 + v1 * v1
            inter = (s0 * s0 - q0) + (s1 * s1 - q1)
            l0 = lin_v[pl.ds(base, 16)]
            l1 = lin_v[pl.ds(base + 16, 16)]
            t = inter * 0.5 + l0 + l1 * lmask
            return jnp.where(lane == i, jnp.sum(t), acc)

        acc = lax.fori_loop(0, _CHUNK, row_body, jnp.zeros((16,), jnp.float32))
        out_v[pl.ds(c * _CHUNK, 16)] = acc
        return 0

    lax.fori_loop(0, _NCHUNK, chunk_body, 0)

    pltpu.sync_copy(out_v, out_hbm.at[pl.ds(wid * _RPW, _RPW)])


@jax.jit
def _fm_sc(x_hi, x_full, x_off, lw_flat, emb4):
    mesh = plsc.VectorSubcoreMesh(core_axis_name="c", subcore_axis_name="s")
    return pl.kernel(
        _fm_body,
        out_type=jax.ShapeDtypeStruct((_B,), jnp.float32),
        mesh=mesh,
        compiler_params=pltpu.CompilerParams(
            needs_layout_passes=False, use_tc_tiling_on_sc=True),
        scratch_types=[
            pltpu.VMEM((_GPW, _G), jnp.int32),
            pltpu.VMEM((_GPW, _G), jnp.int32),
            pltpu.VMEM((_IDX_PER_CHUNK, 128), jnp.float32),
            pltpu.VMEM((_IDX_PER_CHUNK + 16,), jnp.float32),
            pltpu.VMEM((_OFFP,), jnp.int32),
            pltpu.VMEM((_RPW,), jnp.float32),
            pltpu.SemaphoreType.DMA,
        ],
    )(x_hi, x_full, x_off, lw_flat, emb4)


def kernel(x, global_bias, linear_weights, interaction_factors):
    xi = x.astype(jnp.int32)
    x_hi = (xi >> 2).reshape(_NW, _GPW, _G)
    x_full = xi.reshape(_NW, _GPW, _G)
    x_off = jnp.pad(
        ((xi & 3) << 5).reshape(_NW, _NCHUNK, _IDX_PER_CHUNK),
        ((0, 0), (0, 0), (0, _OFFP - _IDX_PER_CHUNK)))
    lw_flat = linear_weights.reshape(-1)
    emb4 = interaction_factors.reshape(250000, 128)
    out = _fm_sc(x_hi, x_full, x_off, lw_flat, emb4)
    return out + global_bias[0]
